# Initial kernel scaffold; baseline (speedup 1.0000x reference)
import jax
import jax.numpy as jnp
from jax.experimental import pallas as pl


def kernel(eeg, video, flow, log, pretrain_edge, pretrain_node, W_edge, b_edge, W_node, b_node, Wg1, Wg2, W_dtf, b_dtf, M1w, M1b, bn1g, bn1b, M2w, M2b, bn2g, bn2b, M3w, M3b):
    raise NotImplementedError("write your pallas kernel here")


# speedup vs baseline: 1.4888x; 1.4888x over previous
import jax
import jax.numpy as jnp
from jax.experimental import pallas as pl
from jax.experimental.pallas import tpu as pltpu

N = 32
D = 128
H = 64
BF = jnp.bfloat16
F32 = jnp.float32


def _dot(a, b):
    return jax.lax.dot_general(a, b, (((1,), (0,)), ((), ())),
                               preferred_element_type=F32)


def _bmm(a, b):
    return jax.lax.dot_general(a, b, (((2,), (1,)), ((0,), (0,))),
                               preferred_element_type=F32)


def _main_body(feat_ref, eeg_ref, pe_ref, pn_ref, We_ref, be_ref, Wn_ref,
               bn2_ref, Wg1_ref, Wg2_ref, Wd_ref, bd_ref,
               pers_ref, vid_ref, dep_ref, comb_ref):
    bb32 = eeg_ref.shape[0]
    bb = bb32 // N

    featb = feat_ref[...].astype(BF)
    eeg3 = eeg_ref[...].reshape(bb, N, D)

    edge = jnp.tanh(_dot(featb, We_ref[...]) + be_ref[...])
    e3 = jax.nn.relu(edge).reshape(bb, 2 * N, N)
    asym = e3[:, :N, :] + e3[:, N:, :]
    dc = 1.0 / jnp.sqrt(asym.sum(axis=1, keepdims=True) + 1e-6)
    dr = 1.0 / jnp.sqrt(asym.sum(axis=2, keepdims=True) + 1e-6)
    pmat = dr * asym * dc

    ap = jax.nn.relu(pe_ref[...])
    ap = ap + ap.T
    dinvp = 1.0 / jnp.sqrt(ap.sum(axis=1, keepdims=True) + 1e-6)
    lp = dinvp * ap * dinvp.reshape(1, N)
    lp3 = jnp.broadcast_to(lp[None], (bb, N, N))

    lcat = jnp.concatenate([lp3, pmat], axis=0).astype(BF)
    np3 = jax.lax.dot_general(
        featb, Wn_ref[...], (((1,), (0,)), ((), ())),
        preferred_element_type=F32) + bn2_ref[...][None]
    hcat = jnp.concatenate([eeg3 + pn_ref[...][None], eeg3 + np3], axis=0)

    m1 = _bmm(lcat, hcat.astype(BF))
    h1 = jax.nn.relu(_dot(m1.reshape(2 * bb32, D).astype(BF), Wg1_ref[...]))
    m2 = _bmm(lcat, h1.reshape(2 * bb, N, H).astype(BF))
    h2 = jax.nn.relu(_dot(m2.reshape(2 * bb32, H).astype(BF), Wg2_ref[...]))

    p2 = h2[:bb32]
    g2 = h2[bb32:]
    pers_ref[...] = p2
    vid_ref[...] = g2

    dep_ref[...] = _dot(eeg_ref[...].astype(BF), Wd_ref[...]) + bd_ref[...]

    comb_ref[...] = jnp.concatenate(
        [p2.reshape(bb, N, H).max(axis=1),
         g2.reshape(bb, N, H).max(axis=1)], axis=-1)


def _head_body(c_ref, m1w_ref, m1b_ref, g1_ref, b1_ref, m2w_ref, m2b_ref,
               g2_ref, b2_ref, m3w_ref, m3b_ref, out_ref):
    c = c_ref[...]
    h = jax.nn.relu(_dot(c.astype(BF), m1w_ref[...]) + m1b_ref[...])
    m = jnp.mean(h, axis=0, keepdims=True)
    v = jnp.mean((h - m) ** 2, axis=0, keepdims=True)
    h = g1_ref[...] * (h - m) / jnp.sqrt(v + 1e-5) + b1_ref[...]
    h = jax.nn.relu(_dot(h.astype(BF), m2w_ref[...]) + m2b_ref[...])
    m = jnp.mean(h, axis=0, keepdims=True)
    v = jnp.mean((h - m) ** 2, axis=0, keepdims=True)
    h = g2_ref[...] * (h - m) / jnp.sqrt(v + 1e-5) + b2_ref[...]
    out_ref[...] = _dot(h.astype(BF), m3w_ref[...]) + m3b_ref[...]


def kernel(eeg, video, flow, log, pretrain_edge, pretrain_node, W_edge,
           b_edge, W_node, b_node, Wg1, Wg2, W_dtf, b_dtf, M1w, M1b, bn1g,
           bn1b, M2w, M2b, bn2g, bn2b, M3w, M3b):
    B = eeg.shape[0]
    K = video.shape[1] + flow.shape[1] + log.shape[1]
    feat = jnp.concatenate([video, flow, log], axis=-1)
    eeg2 = eeg.reshape(B * N, D)
    we_t = W_edge.reshape(K, N, N).transpose(0, 2, 1).reshape(K, N * N)
    we_cat = jnp.concatenate([W_edge, we_t], axis=-1).astype(BF)
    be_t = b_edge.reshape(N, N).T.reshape(N * N)
    be_cat = jnp.concatenate([b_edge, be_t]).reshape(1, 2 * N * N)
    wn3 = W_node.reshape(K, N, D).astype(BF)
    bn2d = b_node.reshape(N, D)
    bd2 = b_dtf.reshape(1, H)

    BB = 128 if B % 128 == 0 else B
    grid = (B // BB,)

    def row(i):
        return (i, 0)

    def fixed(i):
        return (0, 0)

    pers2, vid2, dep2, comb = pl.pallas_call(
        _main_body,
        grid=grid,
        in_specs=[
            pl.BlockSpec((BB, K), row),
            pl.BlockSpec((BB * N, D), row),
            pl.BlockSpec((N, N), fixed),
            pl.BlockSpec((N, D), fixed),
            pl.BlockSpec((K, 2 * N * N), fixed),
            pl.BlockSpec((1, 2 * N * N), fixed),
            pl.BlockSpec((K, N, D), lambda i: (0, 0, 0)),
            pl.BlockSpec((N, D), fixed),
            pl.BlockSpec((D, H), fixed),
            pl.BlockSpec((H, H), fixed),
            pl.BlockSpec((D, H), fixed),
            pl.BlockSpec((1, H), fixed),
        ],
        out_specs=[
            pl.BlockSpec((BB * N, H), row),
            pl.BlockSpec((BB * N, H), row),
            pl.BlockSpec((BB * N, H), row),
            pl.BlockSpec((BB, 2 * H), row),
        ],
        out_shape=[
            jax.ShapeDtypeStruct((B * N, H), F32),
            jax.ShapeDtypeStruct((B * N, H), F32),
            jax.ShapeDtypeStruct((B * N, H), F32),
            jax.ShapeDtypeStruct((B, 2 * H), F32),
        ],
    )(feat, eeg2, pretrain_edge, pretrain_node, we_cat, be_cat, wn3, bn2d,
      Wg1.astype(BF), Wg2.astype(BF), W_dtf.astype(BF), bd2)

    out = pl.pallas_call(
        _head_body,
        out_shape=jax.ShapeDtypeStruct((B, M3w.shape[1]), F32),
    )(comb, M1w.astype(BF), M1b.reshape(1, -1), bn1g.reshape(1, -1),
      bn1b.reshape(1, -1), M2w.astype(BF), M2b.reshape(1, -1),
      bn2g.reshape(1, -1), bn2b.reshape(1, -1), M3w.astype(BF),
      M3b.reshape(1, -1))

    return (out, pers2.reshape(B, N, H), vid2.reshape(B, N, H),
            dep2.reshape(B, N, H))


# speedup vs baseline: 1.5130x; 1.0162x over previous
import jax
import jax.numpy as jnp
from jax.experimental import pallas as pl
from jax.experimental.pallas import tpu as pltpu

N = 32
D = 128
H = 64
BF = jnp.bfloat16
F32 = jnp.float32


def _dot(a, b):
    return jax.lax.dot_general(a, b, (((1,), (0,)), ((), ())),
                               preferred_element_type=F32)


def _bmm(a, b):
    return jax.lax.dot_general(a, b, (((2,), (1,)), ((0,), (0,))),
                               preferred_element_type=F32)


def _main_body(v_ref, f_ref, l_ref, eeg_ref, pe_ref, pn_ref, Wer_ref, be_ref,
               Wnr_ref, bn2_ref, Wg1_ref, Wg2_ref, Wd_ref, bd_ref,
               pers_ref, vid_ref, dep_ref, comb_ref, wec_ref, wnb_ref):
    bb32 = eeg_ref.shape[0]
    bb = bb32 // N
    kd = Wer_ref.shape[0]

    @pl.when(pl.program_id(0) == 0)
    def _():
        wnb_ref[...] = Wnr_ref[...].astype(BF)
        we_raw = Wer_ref[...]
        wec_ref[:, :N * N] = we_raw.astype(BF)
        wet = jnp.swapaxes(we_raw.reshape(kd, N, N), 1, 2)
        wec_ref[:, N * N:] = wet.reshape(kd, N * N).astype(BF)

    featb = jnp.concatenate(
        [v_ref[...], f_ref[...], l_ref[...]], axis=-1).astype(BF)
    eeg3 = eeg_ref[...].reshape(bb, N, D)

    edge = jnp.tanh(_dot(featb, wec_ref[...]) + be_ref[...])
    e3 = jax.nn.relu(edge).reshape(bb, 2 * N, N)
    asym = e3[:, :N, :] + e3[:, N:, :]
    dc = 1.0 / jnp.sqrt(asym.sum(axis=1, keepdims=True) + 1e-6)
    dr = 1.0 / jnp.sqrt(asym.sum(axis=2, keepdims=True) + 1e-6)
    pmat = dr * asym * dc

    ap = jax.nn.relu(pe_ref[...])
    ap = ap + ap.T
    dinvp = 1.0 / jnp.sqrt(ap.sum(axis=1, keepdims=True) + 1e-6)
    lp = dinvp * ap * dinvp.reshape(1, N)
    lp3 = jnp.broadcast_to(lp[None], (bb, N, N))

    lcat = jnp.concatenate([lp3, pmat], axis=0).astype(BF)
    np3 = jax.lax.dot_general(
        featb, wnb_ref[...], (((1,), (0,)), ((), ())),
        preferred_element_type=F32) + bn2_ref[...][None]
    hcat = jnp.concatenate([eeg3 + pn_ref[...][None], eeg3 + np3], axis=0)

    m1 = _bmm(lcat, hcat.astype(BF))
    h1 = jax.nn.relu(_dot(m1.reshape(2 * bb32, D).astype(BF),
                          Wg1_ref[...].astype(BF)))
    m2 = _bmm(lcat, h1.reshape(2 * bb, N, H).astype(BF))
    h2 = jax.nn.relu(_dot(m2.reshape(2 * bb32, H).astype(BF),
                          Wg2_ref[...].astype(BF)))

    p2 = h2[:bb32]
    g2 = h2[bb32:]
    pers_ref[...] = p2
    vid_ref[...] = g2

    dep_ref[...] = _dot(eeg_ref[...].astype(BF),
                        Wd_ref[...].astype(BF)) + bd_ref[...]

    comb_ref[...] = jnp.concatenate(
        [p2.reshape(bb, N, H).max(axis=1),
         g2.reshape(bb, N, H).max(axis=1)], axis=-1)


def _head_body(c_ref, m1w_ref, m1b_ref, g1_ref, b1_ref, m2w_ref, m2b_ref,
               g2_ref, b2_ref, m3w_ref, m3b_ref, out_ref):
    c = c_ref[...]
    h = jax.nn.relu(_dot(c.astype(BF), m1w_ref[...].astype(BF)) + m1b_ref[...])
    m = jnp.mean(h, axis=0, keepdims=True)
    v = jnp.mean((h - m) ** 2, axis=0, keepdims=True)
    h = g1_ref[...] * (h - m) / jnp.sqrt(v + 1e-5) + b1_ref[...]
    h = jax.nn.relu(_dot(h.astype(BF), m2w_ref[...].astype(BF)) + m2b_ref[...])
    m = jnp.mean(h, axis=0, keepdims=True)
    v = jnp.mean((h - m) ** 2, axis=0, keepdims=True)
    h = g2_ref[...] * (h - m) / jnp.sqrt(v + 1e-5) + b2_ref[...]
    out_ref[...] = _dot(h.astype(BF), m3w_ref[...].astype(BF)) + m3b_ref[...]


def kernel(eeg, video, flow, log, pretrain_edge, pretrain_node, W_edge,
           b_edge, W_node, b_node, Wg1, Wg2, W_dtf, b_dtf, M1w, M1b, bn1g,
           bn1b, M2w, M2b, bn2g, bn2b, M3w, M3b):
    B = eeg.shape[0]
    K = video.shape[1] + flow.shape[1] + log.shape[1]
    eeg2 = eeg.reshape(B * N, D)
    be_t = b_edge.reshape(N, N).T.reshape(N * N)
    be_cat = jnp.concatenate([b_edge, be_t]).reshape(1, 2 * N * N)
    wn3 = W_node.reshape(K, N, D)
    bn2d = b_node.reshape(N, D)
    bd2 = b_dtf.reshape(1, H)

    BB = 128 if B % 128 == 0 else B
    grid = (B // BB,)

    def row(i):
        return (i, 0)

    def fixed(i):
        return (0, 0)

    pers2, vid2, dep2, comb = pl.pallas_call(
        _main_body,
        grid=grid,
        in_specs=[
            pl.BlockSpec((BB, video.shape[1]), row),
            pl.BlockSpec((BB, flow.shape[1]), row),
            pl.BlockSpec((BB, log.shape[1]), row),
            pl.BlockSpec((BB * N, D), row),
            pl.BlockSpec((N, N), fixed),
            pl.BlockSpec((N, D), fixed),
            pl.BlockSpec((K, N * N), fixed),
            pl.BlockSpec((1, 2 * N * N), fixed),
            pl.BlockSpec((K, N, D), lambda i: (0, 0, 0)),
            pl.BlockSpec((N, D), fixed),
            pl.BlockSpec((D, H), fixed),
            pl.BlockSpec((H, H), fixed),
            pl.BlockSpec((D, H), fixed),
            pl.BlockSpec((1, H), fixed),
        ],
        scratch_shapes=[pltpu.VMEM((K, 2 * N * N), BF),
                        pltpu.VMEM((K, N, D), BF)],
        out_specs=[
            pl.BlockSpec((BB * N, H), row),
            pl.BlockSpec((BB * N, H), row),
            pl.BlockSpec((BB * N, H), row),
            pl.BlockSpec((BB, 2 * H), row),
        ],
        out_shape=[
            jax.ShapeDtypeStruct((B * N, H), F32),
            jax.ShapeDtypeStruct((B * N, H), F32),
            jax.ShapeDtypeStruct((B * N, H), F32),
            jax.ShapeDtypeStruct((B, 2 * H), F32),
        ],
    )(video, flow, log, eeg2, pretrain_edge, pretrain_node, W_edge, be_cat,
      wn3, bn2d, Wg1, Wg2, W_dtf, bd2)

    out = pl.pallas_call(
        _head_body,
        out_shape=jax.ShapeDtypeStruct((B, M3w.shape[1]), F32),
    )(comb, M1w, M1b.reshape(1, -1), bn1g.reshape(1, -1),
      bn1b.reshape(1, -1), M2w, M2b.reshape(1, -1),
      bn2g.reshape(1, -1), bn2b.reshape(1, -1), M3w,
      M3b.reshape(1, -1))

    return (out, pers2.reshape(B, N, H), vid2.reshape(B, N, H),
            dep2.reshape(B, N, H))
